# baseline (speedup 1.0000x reference)
import jax
import jax.numpy as jnp
import numpy as np
from jax.experimental import pallas as pl
from jax.experimental.pallas import tpu as pltpu

RES_V, RES_H = 16, 32
N_SAMPLES = 100
GRID = 256
N_CH = 11
N_RAYS = RES_V * RES_H
N_PTS = N_RAYS * N_SAMPLES

CHUNK = 1024
N_CHUNKS = N_PTS // CHUNK
N_CHUNKS_PER_CORE = N_CHUNKS // 2


def _dirs_np():
    v = np.arange(RES_V, dtype=np.float32)
    u = np.arange(RES_H, dtype=np.float32)
    v_grid, u_grid = np.meshgrid(v, u, indexing="ij")
    phi = 2.0 * np.pi * u_grid / RES_H
    theta = np.pi * v_grid / RES_V
    st = np.sin(theta)
    dirs = np.stack([st * np.cos(phi), np.cos(theta), st * np.sin(phi)], axis=-1)
    n = np.linalg.norm(dirs, axis=-1, keepdims=True)
    return (dirs / np.maximum(n, 1e-12)).astype(np.float32)


def _gather_body(z_ref, y_ref, x_ref, vol_ref, out_ref, sem):
    i = pl.program_id(0)
    j = pl.program_id(1)
    base = (i * N_CHUNKS_PER_CORE + j) * CHUNK

    def issue(k, carry):
        z = z_ref[0, 0, k]
        y = y_ref[0, 0, k]
        x = x_ref[0, 0, k]
        pltpu.make_async_copy(
            vol_ref.at[:, pl.ds(z, 2), pl.ds(y, 2), pl.ds(x, 2), :, :],
            out_ref.at[base + k],

            sem,
        ).start()
        return carry

    jax.lax.fori_loop(0, CHUNK, issue, 0)
    for _ in range(CHUNK):
        pltpu.make_async_copy(
            vol_ref.at[:, pl.ds(0, 2), pl.ds(0, 2), pl.ds(0, 2), :, :],
            out_ref.at[0],
            sem,
        ).wait()


def _composite_body(val_ref, w_ref, dirs_ref, out_ref):
    acc = jnp.zeros((N_CH, N_RAYS), dtype=jnp.float32)
    trans = jnp.ones((1, N_RAYS), dtype=jnp.float32)
    for t in range(N_SAMPLES):
        v = val_ref[t]
        w = w_ref[t]
        samp = jnp.sum(v * w, axis=0)
        alpha = samp[3:4]
        trans = trans * (1.0 - alpha + 1e-10)
        acc = acc + (alpha * trans) * samp
    dirs = dirs_ref[...]
    s_dot = dirs[0] * acc[8] + dirs[1] * acc[9] + dirs[2] * acc[10]
    env = acc[0:3] + acc[4:7] * jnp.exp(acc[7] * (s_dot - 1.0))[None, :]
    out_ref[0:3] = env
    out_ref[3:8] = jnp.zeros((5, N_RAYS), dtype=jnp.float32)


def kernel(origin, SGLV, voxel_range):
    dirs = jnp.asarray(_dirs_np())
    inf = jnp.float32(np.inf)
    zero = dirs == 0.0
    d_safe = jnp.where(zero, 1.0, dirs)

    def slab(bound):
        num = jnp.broadcast_to(bound - origin, dirs.shape)
        return jnp.where(zero, jnp.where(num > 0, inf, -inf), num / d_safe)

    t_min = slab(voxel_range[0])
    t_max = slab(voxel_range[1])
    t0 = jnp.min(jnp.where(t_min > 0, t_min, inf), axis=-1)
    t1 = jnp.min(jnp.where(t_max > 0, t_max, inf), axis=-1)
    t_end = jnp.minimum(t0, t1)

    lin = jnp.linspace(0.0, 1.0, N_SAMPLES, dtype=jnp.float32)
    ts = lin * t_end[..., None]
    points = origin + ts[..., None] * dirs[:, :, None, :]
    npts = (points - voxel_range[0]) / (voxel_range[1] - voxel_range[0]) * 2.0 - 1.0

    p = npts.reshape(-1, 3)
    ix = (p[:, 0] + 1.0) * 0.5 * (GRID - 1)
    iy = (p[:, 1] + 1.0) * 0.5 * (GRID - 1)
    iz = (p[:, 2] + 1.0) * 0.5 * (GRID - 1)
    vx = jnp.clip(jnp.floor(ix), 0, GRID - 2).astype(jnp.int32)
    vy = jnp.clip(jnp.floor(iy), 0, GRID - 2).astype(jnp.int32)
    vz = jnp.clip(jnp.floor(iz), 0, GRID - 2).astype(jnp.int32)

    def tentw(i, v):
        w0 = jnp.maximum(0.0, 1.0 - jnp.abs(i - v))
        w1 = jnp.maximum(0.0, 1.0 - jnp.abs(i - (v + 1.0)))
        return w0, w1

    wx0, wx1 = tentw(ix, vx.astype(jnp.float32))
    wy0, wy1 = tentw(iy, vy.astype(jnp.float32))
    wz0, wz1 = tentw(iz, vz.astype(jnp.float32))
    w8 = jnp.stack([
        wz0 * wy0 * wx0, wz0 * wy0 * wx1, wz0 * wy1 * wx0, wz0 * wy1 * wx1,
        wz1 * wy0 * wx0, wz1 * wy0 * wx1, wz1 * wy1 * wx0, wz1 * wy1 * wx1,
    ], axis=1)

    idx_shape = (N_CHUNKS, 1, CHUNK)
    zs = vz.reshape(idx_shape)
    ys = vy.reshape(idx_shape)
    xs = vx.reshape(idx_shape)

    grid = (2, N_CHUNKS_PER_CORE)
    idx_spec = pl.BlockSpec(
        (1, 1, CHUNK), lambda i, j: (i * N_CHUNKS_PER_CORE + j, 0, 0),
        memory_space=pltpu.SMEM)
    val8 = pl.pallas_call(
        _gather_body,
        grid=grid,
        in_specs=[idx_spec, idx_spec, idx_spec,
                  pl.BlockSpec(memory_space=pl.ANY)],
        out_specs=pl.BlockSpec(memory_space=pl.ANY),
        out_shape=jax.ShapeDtypeStruct((N_PTS, N_CH, 2, 2, 2, 1, 1), jnp.float32),
        scratch_shapes=[pltpu.SemaphoreType.DMA],
        compiler_params=pltpu.CompilerParams(
            dimension_semantics=("parallel", "arbitrary")),
    )(zs, ys, xs, SGLV.reshape(N_CH, GRID, GRID, GRID, 1, 1))

    val8_t = jnp.transpose(
        val8.reshape(N_RAYS, N_SAMPLES, N_CH, 8), (1, 3, 2, 0))
    w8_t = jnp.transpose(
        w8.reshape(N_RAYS, N_SAMPLES, 8), (1, 2, 0)).reshape(
        N_SAMPLES, 8, 1, N_RAYS)
    dirs8 = jnp.pad(dirs.reshape(N_RAYS, 3).T, ((0, 5), (0, 0)))

    out = pl.pallas_call(
        _composite_body,
        out_shape=jax.ShapeDtypeStruct((8, N_RAYS), jnp.float32),
    )(val8_t, w8_t, dirs8)
    return out[:3].reshape(3, RES_V, RES_H)


# speedup vs baseline: 85.3206x; 85.3206x over previous
import jax
import jax.numpy as jnp
import numpy as np
from jax.experimental import pallas as pl
from jax.experimental.pallas import tpu as pltpu

RES_V, RES_H = 16, 32
N_SAMPLES = 100
GRID = 256
N_CH = 11
N_RAYS = RES_V * RES_H
N_PTS = N_RAYS * N_SAMPLES

CHUNK = 1024
N_CHUNKS = N_PTS // CHUNK
N_CHUNKS_PER_CORE = N_CHUNKS // 2


def _dirs_np():
    v = np.arange(RES_V, dtype=np.float32)
    u = np.arange(RES_H, dtype=np.float32)
    v_grid, u_grid = np.meshgrid(v, u, indexing="ij")
    phi = 2.0 * np.pi * u_grid / RES_H
    theta = np.pi * v_grid / RES_V
    st = np.sin(theta)
    dirs = np.stack([st * np.cos(phi), np.cos(theta), st * np.sin(phi)], axis=-1)
    n = np.linalg.norm(dirs, axis=-1, keepdims=True)
    return (dirs / np.maximum(n, 1e-12)).astype(np.float32)


def _gather_body(z_ref, y_ref, x_ref, volp_ref, out_ref, sem):
    def issue(i, carry):
        for u in range(8):
            k = i * 8 + u
            z = z_ref[0, 0, k]
            y = y_ref[0, 0, k]
            x = x_ref[0, 0, k]
            pltpu.make_async_copy(
                volp_ref.at[pl.ds(z, 2), pl.ds(y, 2), pl.ds(x, 2)],
                out_ref.at[k],
                sem,
            ).start()
        return carry

    jax.lax.fori_loop(0, CHUNK // 8, issue, 0)
    for _ in range(CHUNK):
        pltpu.make_async_copy(
            volp_ref.at[pl.ds(0, 2), pl.ds(0, 2), pl.ds(0, 2)],
            out_ref.at[0],
            sem,
        ).wait()


def _composite_body(val_ref, w_ref, dirs_ref, out_ref):
    acc = jnp.zeros((N_CH, N_RAYS), dtype=jnp.float32)
    trans = jnp.ones((1, N_RAYS), dtype=jnp.float32)
    for t in range(N_SAMPLES):
        v = val_ref[t]
        w = w_ref[t]
        samp = jnp.sum(v * w, axis=0)
        alpha = samp[3:4]
        trans = trans * (1.0 - alpha + 1e-10)
        acc = acc + (alpha * trans) * samp
    dirs = dirs_ref[...]
    s_dot = dirs[0] * acc[8] + dirs[1] * acc[9] + dirs[2] * acc[10]
    env = acc[0:3] + acc[4:7] * jnp.exp(acc[7] * (s_dot - 1.0))[None, :]
    out_ref[0:3] = env
    out_ref[3:8] = jnp.zeros((5, N_RAYS), dtype=jnp.float32)


def kernel(origin, SGLV, voxel_range):
    dirs = jnp.asarray(_dirs_np())
    inf = jnp.float32(np.inf)
    zero = dirs == 0.0
    d_safe = jnp.where(zero, 1.0, dirs)

    def slab(bound):
        num = jnp.broadcast_to(bound - origin, dirs.shape)
        return jnp.where(zero, jnp.where(num > 0, inf, -inf), num / d_safe)

    t_min = slab(voxel_range[0])
    t_max = slab(voxel_range[1])
    t0 = jnp.min(jnp.where(t_min > 0, t_min, inf), axis=-1)
    t1 = jnp.min(jnp.where(t_max > 0, t_max, inf), axis=-1)
    t_end = jnp.minimum(t0, t1)

    lin = jnp.linspace(0.0, 1.0, N_SAMPLES, dtype=jnp.float32)
    ts = lin * t_end[..., None]
    points = origin + ts[..., None] * dirs[:, :, None, :]
    npts = (points - voxel_range[0]) / (voxel_range[1] - voxel_range[0]) * 2.0 - 1.0

    p = npts.reshape(-1, 3)
    ix = (p[:, 0] + 1.0) * 0.5 * (GRID - 1)
    iy = (p[:, 1] + 1.0) * 0.5 * (GRID - 1)
    iz = (p[:, 2] + 1.0) * 0.5 * (GRID - 1)
    vx = jnp.clip(jnp.floor(ix), 0, GRID - 2).astype(jnp.int32)
    vy = jnp.clip(jnp.floor(iy), 0, GRID - 2).astype(jnp.int32)
    vz = jnp.clip(jnp.floor(iz), 0, GRID - 2).astype(jnp.int32)

    def tentw(i, v):
        w0 = jnp.maximum(0.0, 1.0 - jnp.abs(i - v))
        w1 = jnp.maximum(0.0, 1.0 - jnp.abs(i - (v + 1.0)))
        return w0, w1

    wx0, wx1 = tentw(ix, vx.astype(jnp.float32))
    wy0, wy1 = tentw(iy, vy.astype(jnp.float32))
    wz0, wz1 = tentw(iz, vz.astype(jnp.float32))
    w8 = jnp.stack([
        wz0 * wy0 * wx0, wz0 * wy0 * wx1, wz0 * wy1 * wx0, wz0 * wy1 * wx1,
        wz1 * wy0 * wx0, wz1 * wy0 * wx1, wz1 * wy1 * wx0, wz1 * wy1 * wx1,
    ], axis=1)

    volp = jnp.transpose(SGLV.reshape(N_CH, -1), (1, 0)).reshape(
        GRID, GRID, GRID, 1, N_CH)

    idx_shape = (N_CHUNKS, 1, CHUNK)
    zs = vz.reshape(idx_shape)
    ys = vy.reshape(idx_shape)
    xs = vx.reshape(idx_shape)

    grid = (2, N_CHUNKS_PER_CORE)
    idx_spec = pl.BlockSpec(
        (1, 1, CHUNK), lambda i, j: (i * N_CHUNKS_PER_CORE + j, 0, 0),
        memory_space=pltpu.SMEM)
    val8 = pl.pallas_call(
        _gather_body,
        grid=grid,
        in_specs=[idx_spec, idx_spec, idx_spec,
                  pl.BlockSpec(memory_space=pl.ANY)],
        out_specs=pl.BlockSpec(
            (CHUNK, 2, 2, 2, 1, N_CH),
            lambda i, j: (i * N_CHUNKS_PER_CORE + j, 0, 0, 0, 0, 0)),
        out_shape=jax.ShapeDtypeStruct((N_PTS, 2, 2, 2, 1, N_CH), jnp.float32),
        scratch_shapes=[pltpu.SemaphoreType.DMA],
        compiler_params=pltpu.CompilerParams(
            dimension_semantics=("parallel", "arbitrary"),
            disable_bounds_checks=True),
    )(zs, ys, xs, volp)

    val8_t = jnp.transpose(
        val8.reshape(N_RAYS, N_SAMPLES, 8, N_CH), (1, 2, 3, 0))
    w8_t = jnp.transpose(
        w8.reshape(N_RAYS, N_SAMPLES, 8), (1, 2, 0)).reshape(
        N_SAMPLES, 8, 1, N_RAYS)
    dirs8 = jnp.pad(dirs.reshape(N_RAYS, 3).T, ((0, 5), (0, 0)))

    out = pl.pallas_call(
        _composite_body,
        out_shape=jax.ShapeDtypeStruct((8, N_RAYS), jnp.float32),
    )(val8_t, w8_t, dirs8)
    return out[:3].reshape(3, RES_V, RES_H)


# speedup vs baseline: 85.6890x; 1.0043x over previous
import jax
import jax.numpy as jnp
import numpy as np
from jax.experimental import pallas as pl
from jax.experimental.pallas import tpu as pltpu

RES_V, RES_H = 16, 32
N_SAMPLES = 100
GRID = 256
N_CH = 11
N_RAYS = RES_V * RES_H
N_PTS = N_RAYS * N_SAMPLES

CHUNK = 1600
N_CHUNKS = N_PTS // CHUNK
N_CHUNKS_PER_CORE = N_CHUNKS // 2


def _dirs_np():
    v = np.arange(RES_V, dtype=np.float32)
    u = np.arange(RES_H, dtype=np.float32)
    v_grid, u_grid = np.meshgrid(v, u, indexing="ij")
    phi = 2.0 * np.pi * u_grid / RES_H
    theta = np.pi * v_grid / RES_V
    st = np.sin(theta)
    dirs = np.stack([st * np.cos(phi), np.cos(theta), st * np.sin(phi)], axis=-1)
    n = np.linalg.norm(dirs, axis=-1, keepdims=True)
    return (dirs / np.maximum(n, 1e-12)).astype(np.float32)


def _gather_body(z_ref, y_ref, x_ref, volp_ref, out_ref, sem):
    def issue(i, carry):
        for u in range(8):
            k = i * 8 + u
            z = z_ref[0, 0, k]
            y = y_ref[0, 0, k]
            x = x_ref[0, 0, k]
            pltpu.make_async_copy(
                volp_ref.at[pl.ds(z, 2), pl.ds(y, 2), pl.ds(x, 2)],
                out_ref.at[k],
                sem,
            ).start()
        return carry

    jax.lax.fori_loop(0, CHUNK // 8, issue, 0)
    for _ in range(CHUNK):
        pltpu.make_async_copy(
            volp_ref.at[pl.ds(0, 2), pl.ds(0, 2), pl.ds(0, 2)],
            out_ref.at[0],
            sem,
        ).wait()


def _composite_body(val_ref, w_ref, dirs_ref, out_ref):
    acc = jnp.zeros((N_CH, N_RAYS), dtype=jnp.float32)
    trans = jnp.ones((1, N_RAYS), dtype=jnp.float32)
    for t in range(N_SAMPLES):
        v = val_ref[t]
        w = w_ref[t]
        samp = jnp.sum(v * w, axis=0)
        alpha = samp[3:4]
        trans = trans * (1.0 - alpha + 1e-10)
        acc = acc + (alpha * trans) * samp
    dirs = dirs_ref[...]
    s_dot = dirs[0] * acc[8] + dirs[1] * acc[9] + dirs[2] * acc[10]
    env = acc[0:3] + acc[4:7] * jnp.exp(acc[7] * (s_dot - 1.0))[None, :]
    out_ref[0:3] = env
    out_ref[3:8] = jnp.zeros((5, N_RAYS), dtype=jnp.float32)


def kernel(origin, SGLV, voxel_range):
    dirs = jnp.asarray(_dirs_np())
    inf = jnp.float32(np.inf)
    zero = dirs == 0.0
    d_safe = jnp.where(zero, 1.0, dirs)

    def slab(bound):
        num = jnp.broadcast_to(bound - origin, dirs.shape)
        return jnp.where(zero, jnp.where(num > 0, inf, -inf), num / d_safe)

    t_min = slab(voxel_range[0])
    t_max = slab(voxel_range[1])
    t0 = jnp.min(jnp.where(t_min > 0, t_min, inf), axis=-1)
    t1 = jnp.min(jnp.where(t_max > 0, t_max, inf), axis=-1)
    t_end = jnp.minimum(t0, t1)

    lin = jnp.linspace(0.0, 1.0, N_SAMPLES, dtype=jnp.float32)
    ts = lin * t_end[..., None]
    points = origin + ts[..., None] * dirs[:, :, None, :]
    npts = (points - voxel_range[0]) / (voxel_range[1] - voxel_range[0]) * 2.0 - 1.0

    p = npts.reshape(-1, 3)
    ix = (p[:, 0] + 1.0) * 0.5 * (GRID - 1)
    iy = (p[:, 1] + 1.0) * 0.5 * (GRID - 1)
    iz = (p[:, 2] + 1.0) * 0.5 * (GRID - 1)
    vx = jnp.clip(jnp.floor(ix), 0, GRID - 2).astype(jnp.int32)
    vy = jnp.clip(jnp.floor(iy), 0, GRID - 2).astype(jnp.int32)
    vz = jnp.clip(jnp.floor(iz), 0, GRID - 2).astype(jnp.int32)

    def tentw(i, v):
        w0 = jnp.maximum(0.0, 1.0 - jnp.abs(i - v))
        w1 = jnp.maximum(0.0, 1.0 - jnp.abs(i - (v + 1.0)))
        return w0, w1

    wx0, wx1 = tentw(ix, vx.astype(jnp.float32))
    wy0, wy1 = tentw(iy, vy.astype(jnp.float32))
    wz0, wz1 = tentw(iz, vz.astype(jnp.float32))
    w8 = jnp.stack([
        wz0 * wy0 * wx0, wz0 * wy0 * wx1, wz0 * wy1 * wx0, wz0 * wy1 * wx1,
        wz1 * wy0 * wx0, wz1 * wy0 * wx1, wz1 * wy1 * wx0, wz1 * wy1 * wx1,
    ], axis=1)

    volp = jnp.transpose(SGLV.reshape(N_CH, -1), (1, 0)).reshape(
        GRID, GRID, GRID, 1, N_CH)

    idx_shape = (N_CHUNKS, 1, CHUNK)
    zs = vz.reshape(idx_shape)
    ys = vy.reshape(idx_shape)
    xs = vx.reshape(idx_shape)

    grid = (2, N_CHUNKS_PER_CORE)
    idx_spec = pl.BlockSpec(
        (1, 1, CHUNK), lambda i, j: (i * N_CHUNKS_PER_CORE + j, 0, 0),
        memory_space=pltpu.SMEM)
    val8 = pl.pallas_call(
        _gather_body,
        grid=grid,
        in_specs=[idx_spec, idx_spec, idx_spec,
                  pl.BlockSpec(memory_space=pl.ANY)],
        out_specs=pl.BlockSpec(
            (CHUNK, 2, 2, 2, 1, N_CH),
            lambda i, j: (i * N_CHUNKS_PER_CORE + j, 0, 0, 0, 0, 0)),
        out_shape=jax.ShapeDtypeStruct((N_PTS, 2, 2, 2, 1, N_CH), jnp.float32),
        scratch_shapes=[pltpu.SemaphoreType.DMA],
        compiler_params=pltpu.CompilerParams(
            dimension_semantics=("parallel", "arbitrary"),
            disable_bounds_checks=True),
    )(zs, ys, xs, volp)

    val8_t = jnp.transpose(
        val8.reshape(N_RAYS, N_SAMPLES, 8, N_CH), (1, 2, 3, 0))
    w8_t = jnp.transpose(
        w8.reshape(N_RAYS, N_SAMPLES, 8), (1, 2, 0)).reshape(
        N_SAMPLES, 8, 1, N_RAYS)
    dirs8 = jnp.pad(dirs.reshape(N_RAYS, 3).T, ((0, 5), (0, 0)))

    out = pl.pallas_call(
        _composite_body,
        out_shape=jax.ShapeDtypeStruct((8, N_RAYS), jnp.float32),
    )(val8_t, w8_t, dirs8)
    return out[:3].reshape(3, RES_V, RES_H)
